# Initial kernel scaffold; baseline (speedup 1.0000x reference)
import jax
import jax.numpy as jnp
from jax.experimental import pallas as pl


def kernel(x, edge_index, batch, W1, b1, W2, b2, W0, b0, Wf1, bf1, Wf2, bf2):
    raise NotImplementedError("write your pallas kernel here")


# speedup vs baseline: 28.0994x; 28.0994x over previous
import functools

import jax
import jax.numpy as jnp
from jax import lax
from jax.experimental import pallas as pl
from jax.experimental.pallas import tpu as pltpu
from jax.experimental.pallas import tpu_sc as plsc

N = 100000
E = 3200000
G = 1024
GP = G + 128
NP = 102400
EROWS = 25600
E2 = EROWS * 128
BROWS = 896
NC = 2
NS = 16
NW = NC * NS
ROWS_PER_W = EROWS // NW
CHR = 8
NCHUNK = ROWS_PER_W // CHR
BR_PER_T = BROWS // NS

_mesh = plsc.VectorSubcoreMesh(core_axis_name="c", subcore_axis_name="s")
_f32 = jnp.float32
_i32 = jnp.int32


def _k1_body(dst_hbm, batch_hbm, zn_hbm, degout, cntout,
             idx_v, bidx_v, ones_v, sem, sh_deg, sh_cnt):
    c = lax.axis_index("c")
    s = lax.axis_index("s")
    wid = c * NS + s

    def _o(i, _):
        ones_v[pl.ds(i * 16, 16)] = jnp.ones((16,), _f32)
        return 0
    lax.fori_loop(0, 8, _o, 0)

    @pl.when(s == 0)
    def _():
        pltpu.sync_copy(zn_hbm, sh_deg)

    @pl.when(jnp.logical_and(c == 0, s == 1))
    def _():
        pltpu.sync_copy(zn_hbm.at[pl.ds(0, GP)], sh_cnt)

    plsc.subcore_barrier()

    def _chunk(ch, _):
        base = wid * ROWS_PER_W + ch * CHR
        pltpu.sync_copy(dst_hbm.at[pl.ds(base, CHR)], idx_v)
        ds = []
        for j in range(CHR):
            ds.append(pltpu.async_copy(
                ones_v, sh_deg.at[idx_v.at[j]], sem, add=True))
        for d in ds:
            d.wait()
        return 0
    lax.fori_loop(0, NCHUNK, _chunk, 0)

    @pl.when(c == 0)
    def _():
        pltpu.sync_copy(batch_hbm.at[pl.ds(s * BR_PER_T, BR_PER_T)], bidx_v)
        for t in range(BR_PER_T // 4):
            ds = []
            for j in range(4):
                ds.append(pltpu.async_copy(
                    ones_v, sh_cnt.at[bidx_v.at[t * 4 + j]], sem, add=True))
            for d in ds:
                d.wait()

    plsc.subcore_barrier()

    @pl.when(s == 0)
    def _():
        pltpu.sync_copy(sh_deg, degout.at[c])

    @pl.when(jnp.logical_and(c == 0, s == 1))
    def _():
        pltpu.sync_copy(sh_cnt, cntout)


_k1 = functools.partial(
    pl.kernel,
    out_type=(jax.ShapeDtypeStruct((NC, NP), _f32),
              jax.ShapeDtypeStruct((GP,), _f32)),
    mesh=_mesh,
    compiler_params=pltpu.CompilerParams(use_tc_tiling_on_sc=False),
    scratch_types=[
        pltpu.VMEM((CHR, 128), _i32),
        pltpu.VMEM((BR_PER_T, 128), _i32),
        pltpu.VMEM((128,), _f32),
        pltpu.SemaphoreType.DMA,
        pltpu.VMEM_SHARED((NP,), _f32),
        pltpu.VMEM_SHARED((GP,), _f32),
    ],
)(_k1_body)


def _k3_body(hs_hbm, src_hbm, dst_hbm, zn16_hbm, outa, outb,
             idx_s, idx_d, rows_v, gsem, ssem, s_sh):
    c = lax.axis_index("c")
    s = lax.axis_index("s")
    wid = c * NS + s

    @pl.when(s == 0)
    def _():
        pltpu.sync_copy(zn16_hbm, s_sh)

    plsc.subcore_barrier()

    def _chunk(ch, _):
        base = wid * ROWS_PER_W + ch * CHR
        pltpu.sync_copy(src_hbm.at[pl.ds(base, CHR)], idx_s)
        pltpu.sync_copy(dst_hbm.at[pl.ds(base, CHR)], idx_d)
        gd = []
        for j in range(CHR):
            gd.append(pltpu.async_copy(
                hs_hbm.at[idx_s.at[j]],
                rows_v.at[pl.ds(j * 128, 128)], gsem))
        for d in gd:
            d.wait()
        sd = []
        for j in range(CHR):
            sd.append(pltpu.async_copy(
                rows_v.at[pl.ds(j * 128, 128)],
                s_sh.at[idx_d.at[j]], ssem, add=True))
        for d in sd:
            d.wait()
        return 0
    lax.fori_loop(0, NCHUNK, _chunk, 0)
    plsc.subcore_barrier()

    @pl.when(jnp.logical_and(c == 0, s == 0))
    def _():
        pltpu.sync_copy(s_sh, outa)

    @pl.when(jnp.logical_and(c == 1, s == 0))
    def _():
        pltpu.sync_copy(s_sh, outb)


_k3 = functools.partial(
    pl.kernel,
    out_type=(jax.ShapeDtypeStruct((NP, 16), _f32),
              jax.ShapeDtypeStruct((NP, 16), _f32)),
    mesh=_mesh,
    compiler_params=pltpu.CompilerParams(use_tc_tiling_on_sc=False),
    scratch_types=[
        pltpu.VMEM((CHR, 128), _i32),
        pltpu.VMEM((CHR, 128), _i32),
        pltpu.VMEM((CHR * 128, 16), _f32),
        pltpu.SemaphoreType.DMA,
        pltpu.SemaphoreType.DMA,
        pltpu.VMEM_SHARED((NP, 16), _f32),
    ],
)(_k3_body)


def _k5_body(pos_hbm, dinvr_hbm, hs2a_hbm, hs2b_hbm, sa0_h, sa1_h, sb0_h, sb1_h,
             b2_hbm, ga_out, gb_out,
             idxv, dbuf, ba, bb, a0, a1, b0_, b1_, biasv, gav, gbv, sem):
    c = lax.axis_index("c")
    s = lax.axis_index("s")
    wid = c * NS + s
    base = wid * 32
    pltpu.sync_copy(pos_hbm.at[pl.ds(base, 32)], idxv)
    pltpu.sync_copy(b2_hbm, biasv)
    gd = []
    for tbl, buf in ((dinvr_hbm, dbuf), (hs2a_hbm, ba), (hs2b_hbm, bb),
                     (sa0_h, a0), (sa1_h, a1), (sb0_h, b0_), (sb1_h, b1_)):
        gd.append(pltpu.async_copy(tbl.at[idxv], buf, sem))
    for d in gd:
        d.wait()
    b2a = biasv[pl.ds(0, 16)]
    b2b = biasv[pl.ds(16, 16)]
    for r in range(32):
        di = dbuf[r]
        ha = jnp.maximum(di * (ba[r] + a0[r] + a1[r]) + b2a, 0.0)
        hb = jnp.maximum(di * (bb[r] + b0_[r] + b1_[r]) + b2b, 0.0)
        gav[r] = ha
        gbv[r] = hb
    pltpu.sync_copy(gav, ga_out.at[pl.ds(base, 32)])
    pltpu.sync_copy(gbv, gb_out.at[pl.ds(base, 32)])


_k5 = functools.partial(
    pl.kernel,
    out_type=(jax.ShapeDtypeStruct((G, 16), _f32),
              jax.ShapeDtypeStruct((G, 16), _f32)),
    mesh=_mesh,
    compiler_params=pltpu.CompilerParams(use_tc_tiling_on_sc=False),
    scratch_types=[
        pltpu.VMEM((32,), _i32),
        pltpu.VMEM((32, 16), _f32),
        pltpu.VMEM((32, 16), _f32),
        pltpu.VMEM((32, 16), _f32),
        pltpu.VMEM((32, 16), _f32),
        pltpu.VMEM((32, 16), _f32),
        pltpu.VMEM((32, 16), _f32),
        pltpu.VMEM((32, 16), _f32),
        pltpu.VMEM((32,), _f32),
        pltpu.VMEM((32, 16), _f32),
        pltpu.VMEM((32, 16), _f32),
        pltpu.SemaphoreType.DMA,
    ],
)(_k5_body)


_BLK = 1024


def _kpos_body(cnt_ref, pos_ref):
    r = lax.broadcasted_iota(_i32, (G, G), 0)
    col = lax.broadcasted_iota(_i32, (G, G), 1)
    tri = (r <= col).astype(_f32)
    cum = jnp.dot(cnt_ref[...], tri, preferred_element_type=_f32)
    p = cum.astype(_i32) - 1
    pos_ref[...] = jnp.where(p < 0, N - 1, p)


def _kpos(cnt):
    return pl.pallas_call(
        _kpos_body,
        out_shape=jax.ShapeDtypeStruct((1, G), _i32),
    )(cnt)


def _k2_body(x_ref, degt_ref, w1_ref, hs1_ref, dinvr_ref):
    deg = degt_ref[:, 0:1] + degt_ref[:, 1:2] + 1.0
    dinv = lax.rsqrt(deg)
    h = jnp.dot(x_ref[...], w1_ref[...], preferred_element_type=_f32)
    hs1_ref[...] = h * dinv
    dinvr_ref[...] = jnp.broadcast_to(dinv, (_BLK, 16))


def _k2(xp, degt, w1p):
    return pl.pallas_call(
        _k2_body,
        grid=(NP // _BLK,),
        in_specs=[
            pl.BlockSpec((_BLK, 16), lambda i: (i, 0)),
            pl.BlockSpec((_BLK, 2), lambda i: (i, 0)),
            pl.BlockSpec((16, 16), lambda i: (0, 0)),
        ],
        out_specs=[
            pl.BlockSpec((_BLK, 16), lambda i: (i, 0)),
            pl.BlockSpec((_BLK, 16), lambda i: (i, 0)),
        ],
        out_shape=[
            jax.ShapeDtypeStruct((NP, 16), _f32),
            jax.ShapeDtypeStruct((NP, 16), _f32),
        ],
    )(xp, degt, w1p)


def _k4_body(hs1_ref, s1a_ref, s1b_ref, dinvr_ref, b1_ref, w2_ref,
             hs2a_ref, hs2b_ref):
    di = dinvr_ref[...]
    h1 = jnp.maximum(
        di * (hs1_ref[...] + s1a_ref[...] + s1b_ref[...]) + b1_ref[...], 0.0)
    t = jnp.dot(h1, w2_ref[...], preferred_element_type=_f32)
    dc = di[:, 0:1]
    hs2a_ref[...] = t[:, 0:16] * dc
    hs2b_ref[...] = t[:, 16:32] * dc


def _k4(hs1, s1a, s1b, dinvr, b1p, w2p):
    return pl.pallas_call(
        _k4_body,
        grid=(NP // _BLK,),
        in_specs=[
            pl.BlockSpec((_BLK, 16), lambda i: (i, 0)),
            pl.BlockSpec((_BLK, 16), lambda i: (i, 0)),
            pl.BlockSpec((_BLK, 16), lambda i: (i, 0)),
            pl.BlockSpec((_BLK, 16), lambda i: (i, 0)),
            pl.BlockSpec((1, 16), lambda i: (0, 0)),
            pl.BlockSpec((16, 32), lambda i: (0, 0)),
        ],
        out_specs=[
            pl.BlockSpec((_BLK, 16), lambda i: (i, 0)),
            pl.BlockSpec((_BLK, 16), lambda i: (i, 0)),
        ],
        out_shape=[
            jax.ShapeDtypeStruct((NP, 16), _f32),
            jax.ShapeDtypeStruct((NP, 16), _f32),
        ],
    )(hs1, s1a, s1b, dinvr, b1p, w2p)


def _k6_body(ga_ref, gb_ref, w0a_ref, w0b_ref, b0_ref, wf1_ref, bf1_ref,
             wf2_ref, bf2_ref, out_ref):
    t = (jnp.dot(ga_ref[...], w0a_ref[...], preferred_element_type=_f32)
         + jnp.dot(gb_ref[...], w0b_ref[...], preferred_element_type=_f32)
         + b0_ref[...])
    t = jnp.maximum(t, 0.0)
    t = jnp.maximum(
        jnp.dot(t, wf1_ref[...], preferred_element_type=_f32) + bf1_ref[...],
        0.0)
    out_ref[...] = (jnp.dot(t, wf2_ref[...], preferred_element_type=_f32)
                    + bf2_ref[...])


def _k6(ga, gb, w0a, w0b, b0p, wf1p, bf1p, wf2p, bf2r):
    return pl.pallas_call(
        _k6_body,
        out_shape=jax.ShapeDtypeStruct((G, 1), _f32),
    )(ga, gb, w0a, w0b, b0p, wf1p, bf1p, wf2p, bf2r)


def kernel(x, edge_index, batch, W1, b1, W2, b2, W0, b0, Wf1, bf1, Wf2, bf2):
    xp = jnp.pad(x, ((0, NP - N), (0, 5)))
    padidx = jnp.full((E2 - E,), NP - 1, _i32)
    src2 = jnp.concatenate([edge_index[0], padidx]).reshape(EROWS, 128)
    dst2 = jnp.concatenate([edge_index[1], padidx]).reshape(EROWS, 128)
    batch2 = jnp.pad(batch, (0, BROWS * 128 - N),
                     constant_values=G).reshape(BROWS, 128)
    zn = jnp.zeros((NP,), _f32)
    zn16 = jnp.zeros((NP, 16), _f32)
    w1p = jnp.pad(W1, ((0, 5), (0, 1)))
    b1p = jnp.pad(b1, (0, 1)).reshape(1, 16)
    w2p = jnp.pad(W2, ((0, 1), (0, 13)))
    b2ab = jnp.pad(b2, (0, 13))
    w0p = jnp.pad(W0, ((0, 13), (0, 6)))
    w0a, w0b = w0p[:16], w0p[16:]
    b0p = jnp.pad(b0, (0, 6)).reshape(1, 16)
    wf1p = jnp.pad(Wf1, ((0, 6), (0, 11)))
    bf1p = jnp.pad(bf1, (0, 11)).reshape(1, 16)
    wf2p = jnp.pad(Wf2, ((0, 11), (0, 0)))
    bf2r = bf2.reshape(1, 1)

    degpart, cnt = _k1(dst2, batch2, zn)
    pos = _kpos(cnt[:G].reshape(1, G)).reshape(G)
    degt = degpart.T
    hs1, dinvr = _k2(xp, degt, w1p)
    s1a, s1b = _k3(hs1, src2, dst2, zn16)
    hs2a, hs2b = _k4(hs1, s1a, s1b, dinvr, b1p, w2p)
    s2a0, s2a1 = _k3(hs2a, src2, dst2, zn16)
    s2b0, s2b1 = _k3(hs2b, src2, dst2, zn16)
    ga, gb = _k5(pos, dinvr, hs2a, hs2b, s2a0, s2a1, s2b0, s2b1, b2ab)
    return _k6(ga, gb, w0a, w0b, b0p, wf1p, bf1p, wf2p, bf2r)


# speedup vs baseline: 32.6317x; 1.1613x over previous
import functools

import jax
import jax.numpy as jnp
from jax import lax
from jax.experimental import pallas as pl
from jax.experimental.pallas import tpu as pltpu
from jax.experimental.pallas import tpu_sc as plsc

N = 100000
E = 3200000
G = 1024
GP = G + 128
NP = 102400
EROWS = 25600
E2 = EROWS * 128
BROWS = 896
NC = 2
NS = 16
NW = NC * NS
ROWS_PER_W = EROWS // NW
CHR = 5
NCHUNK = ROWS_PER_W // CHR
BR_PER_T = BROWS // NS

_mesh = plsc.VectorSubcoreMesh(core_axis_name="c", subcore_axis_name="s")
_f32 = jnp.float32
_i32 = jnp.int32


def _k1_body(dst_hbm, batch_hbm, zn_hbm, degout, cntout,
             idx_v, bidx_v, ones_v, sem, sh_deg, sh_cnt):
    c = lax.axis_index("c")
    s = lax.axis_index("s")
    wid = c * NS + s

    def _o(i, _):
        ones_v[pl.ds(i * 16, 16)] = jnp.ones((16,), _f32)
        return 0
    lax.fori_loop(0, 8, _o, 0)

    @pl.when(s == 0)
    def _():
        pltpu.sync_copy(zn_hbm, sh_deg)

    @pl.when(jnp.logical_and(c == 0, s == 1))
    def _():
        pltpu.sync_copy(zn_hbm.at[pl.ds(0, GP)], sh_cnt)

    plsc.subcore_barrier()

    def _chunk(ch, _):
        base = wid * ROWS_PER_W + ch * CHR
        pltpu.sync_copy(dst_hbm.at[pl.ds(base, CHR)], idx_v)
        ds = []
        for j in range(CHR):
            ds.append(pltpu.async_copy(
                ones_v, sh_deg.at[idx_v.at[j]], sem, add=True))
        for d in ds:
            d.wait()
        return 0
    lax.fori_loop(0, NCHUNK, _chunk, 0)

    @pl.when(c == 0)
    def _():
        pltpu.sync_copy(batch_hbm.at[pl.ds(s * BR_PER_T, BR_PER_T)], bidx_v)
        for t in range(BR_PER_T // 4):
            ds = []
            for j in range(4):
                ds.append(pltpu.async_copy(
                    ones_v, sh_cnt.at[bidx_v.at[t * 4 + j]], sem, add=True))
            for d in ds:
                d.wait()

    plsc.subcore_barrier()

    @pl.when(s == 0)
    def _():
        pltpu.sync_copy(sh_deg, degout.at[c])

    @pl.when(jnp.logical_and(c == 0, s == 1))
    def _():
        pltpu.sync_copy(sh_cnt, cntout)


_k1 = functools.partial(
    pl.kernel,
    out_type=(jax.ShapeDtypeStruct((NC, NP), _f32),
              jax.ShapeDtypeStruct((GP,), _f32)),
    mesh=_mesh,
    compiler_params=pltpu.CompilerParams(use_tc_tiling_on_sc=False),
    scratch_types=[
        pltpu.VMEM((CHR, 128), _i32),
        pltpu.VMEM((BR_PER_T, 128), _i32),
        pltpu.VMEM((128,), _f32),
        pltpu.SemaphoreType.DMA,
        pltpu.VMEM_SHARED((NP,), _f32),
        pltpu.VMEM_SHARED((GP,), _f32),
    ],
)(_k1_body)


def _k3_body(hs_hbm, src_hbm, dst_hbm, zn16_hbm, outa, outb,
             idx_sa, idx_da, idx_sb, idx_db, rows_a, rows_b,
             gsa, gsb, ssa, ssb, s_sh):
    c = lax.axis_index("c")
    s = lax.axis_index("s")
    wid = c * NS + s

    @pl.when(s == 0)
    def _():
        pltpu.sync_copy(zn16_hbm, s_sh)

    plsc.subcore_barrier()
    w0 = wid * ROWS_PER_W

    def _stage(ch, isx, idx):
        pltpu.sync_copy(src_hbm.at[pl.ds(w0 + ch * CHR, CHR)], isx)
        pltpu.sync_copy(dst_hbm.at[pl.ds(w0 + ch * CHR, CHR)], idx)

    def _gfire(isx, rows, sem):
        for j in range(CHR):
            pltpu.async_copy(hs_hbm.at[isx.at[j]],
                             rows.at[pl.ds(j * 128, 128)], sem)

    def _gwait(isx, rows, sem):
        for j in range(CHR):
            pltpu.make_async_copy(hs_hbm.at[isx.at[j]],
                                  rows.at[pl.ds(j * 128, 128)], sem).wait()

    def _sfire(idx, rows, sem):
        for j in range(CHR):
            pltpu.async_copy(rows.at[pl.ds(j * 128, 128)],
                             s_sh.at[idx.at[j]], sem, add=True)

    def _swait(idx, rows, sem):
        for j in range(CHR):
            pltpu.make_async_copy(rows.at[pl.ds(j * 128, 128)],
                                  s_sh.at[idx.at[j]], sem).wait()

    _stage(0, idx_sa, idx_da)
    _gfire(idx_sa, rows_a, gsa)
    _stage(1, idx_sb, idx_db)
    _gwait(idx_sa, rows_a, gsa)
    _sfire(idx_da, rows_a, ssa)
    _gfire(idx_sb, rows_b, gsb)

    def _steady(ch, _):
        @pl.when(ch % 2 == 0)
        def _():
            _swait(idx_db, rows_b, ssb)
            _stage(ch + 1, idx_sb, idx_db)
            _gwait(idx_sa, rows_a, gsa)
            _gfire(idx_sb, rows_b, gsb)
            _sfire(idx_da, rows_a, ssa)

        @pl.when(ch % 2 == 1)
        def _():
            _swait(idx_da, rows_a, ssa)
            _stage(ch + 1, idx_sa, idx_da)
            _gwait(idx_sb, rows_b, gsb)
            _gfire(idx_sa, rows_a, gsa)
            _sfire(idx_db, rows_b, ssb)
        return 0
    lax.fori_loop(1, NCHUNK - 1, _steady, 0)

    _swait(idx_da, rows_a, ssa)
    _gwait(idx_sb, rows_b, gsb)
    _sfire(idx_db, rows_b, ssb)
    _swait(idx_db, rows_b, ssb)
    plsc.subcore_barrier()

    @pl.when(jnp.logical_and(c == 0, s == 0))
    def _():
        pltpu.sync_copy(s_sh, outa)

    @pl.when(jnp.logical_and(c == 1, s == 0))
    def _():
        pltpu.sync_copy(s_sh, outb)


_k3 = functools.partial(
    pl.kernel,
    out_type=(jax.ShapeDtypeStruct((NP, 16), _f32),
              jax.ShapeDtypeStruct((NP, 16), _f32)),
    mesh=_mesh,
    compiler_params=pltpu.CompilerParams(use_tc_tiling_on_sc=False),
    scratch_types=[
        pltpu.VMEM((CHR, 128), _i32),
        pltpu.VMEM((CHR, 128), _i32),
        pltpu.VMEM((CHR, 128), _i32),
        pltpu.VMEM((CHR, 128), _i32),
        pltpu.VMEM((CHR * 128, 16), _f32),
        pltpu.VMEM((CHR * 128, 16), _f32),
        pltpu.SemaphoreType.DMA,
        pltpu.SemaphoreType.DMA,
        pltpu.SemaphoreType.DMA,
        pltpu.SemaphoreType.DMA,
        pltpu.VMEM_SHARED((NP, 16), _f32),
    ],
)(_k3_body)


def _k5_body(pos_hbm, dinvr_hbm, hs2a_hbm, hs2b_hbm, sa0_h, sa1_h, sb0_h, sb1_h,
             b2_hbm, ga_out, gb_out,
             idxv, dbuf, ba, bb, a0, a1, b0_, b1_, biasv, gav, gbv, sem):
    c = lax.axis_index("c")
    s = lax.axis_index("s")
    wid = c * NS + s
    base = wid * 32
    pltpu.sync_copy(pos_hbm.at[pl.ds(base, 32)], idxv)
    pltpu.sync_copy(b2_hbm, biasv)
    gd = []
    for tbl, buf in ((dinvr_hbm, dbuf), (hs2a_hbm, ba), (hs2b_hbm, bb),
                     (sa0_h, a0), (sa1_h, a1), (sb0_h, b0_), (sb1_h, b1_)):
        gd.append(pltpu.async_copy(tbl.at[idxv], buf, sem))
    for d in gd:
        d.wait()
    b2a = biasv[pl.ds(0, 16)]
    b2b = biasv[pl.ds(16, 16)]
    for r in range(32):
        di = dbuf[r]
        ha = jnp.maximum(di * (ba[r] + a0[r] + a1[r]) + b2a, 0.0)
        hb = jnp.maximum(di * (bb[r] + b0_[r] + b1_[r]) + b2b, 0.0)
        gav[r] = ha
        gbv[r] = hb
    pltpu.sync_copy(gav, ga_out.at[pl.ds(base, 32)])
    pltpu.sync_copy(gbv, gb_out.at[pl.ds(base, 32)])


_k5 = functools.partial(
    pl.kernel,
    out_type=(jax.ShapeDtypeStruct((G, 16), _f32),
              jax.ShapeDtypeStruct((G, 16), _f32)),
    mesh=_mesh,
    compiler_params=pltpu.CompilerParams(use_tc_tiling_on_sc=False),
    scratch_types=[
        pltpu.VMEM((32,), _i32),
        pltpu.VMEM((32, 16), _f32),
        pltpu.VMEM((32, 16), _f32),
        pltpu.VMEM((32, 16), _f32),
        pltpu.VMEM((32, 16), _f32),
        pltpu.VMEM((32, 16), _f32),
        pltpu.VMEM((32, 16), _f32),
        pltpu.VMEM((32, 16), _f32),
        pltpu.VMEM((32,), _f32),
        pltpu.VMEM((32, 16), _f32),
        pltpu.VMEM((32, 16), _f32),
        pltpu.SemaphoreType.DMA,
    ],
)(_k5_body)


_BLK = 1024


def _kpos_body(cnt_ref, pos_ref):
    r = lax.broadcasted_iota(_i32, (G, G), 0)
    col = lax.broadcasted_iota(_i32, (G, G), 1)
    tri = (r <= col).astype(_f32)
    cum = jnp.dot(cnt_ref[...], tri, preferred_element_type=_f32)
    p = cum.astype(_i32) - 1
    pos_ref[...] = jnp.where(p < 0, N - 1, p)


def _kpos(cnt):
    return pl.pallas_call(
        _kpos_body,
        out_shape=jax.ShapeDtypeStruct((1, G), _i32),
    )(cnt)


def _k2_body(x_ref, degt_ref, w1_ref, hs1_ref, dinvr_ref):
    deg = degt_ref[:, 0:1] + degt_ref[:, 1:2] + 1.0
    dinv = lax.rsqrt(deg)
    h = jnp.dot(x_ref[...], w1_ref[...], preferred_element_type=_f32)
    hs1_ref[...] = h * dinv
    dinvr_ref[...] = jnp.broadcast_to(dinv, (_BLK, 16))


def _k2(xp, degt, w1p):
    return pl.pallas_call(
        _k2_body,
        grid=(NP // _BLK,),
        in_specs=[
            pl.BlockSpec((_BLK, 16), lambda i: (i, 0)),
            pl.BlockSpec((_BLK, 2), lambda i: (i, 0)),
            pl.BlockSpec((16, 16), lambda i: (0, 0)),
        ],
        out_specs=[
            pl.BlockSpec((_BLK, 16), lambda i: (i, 0)),
            pl.BlockSpec((_BLK, 16), lambda i: (i, 0)),
        ],
        out_shape=[
            jax.ShapeDtypeStruct((NP, 16), _f32),
            jax.ShapeDtypeStruct((NP, 16), _f32),
        ],
    )(xp, degt, w1p)


def _k4_body(hs1_ref, s1a_ref, s1b_ref, dinvr_ref, b1_ref, w2_ref,
             hs2a_ref, hs2b_ref):
    di = dinvr_ref[...]
    h1 = jnp.maximum(
        di * (hs1_ref[...] + s1a_ref[...] + s1b_ref[...]) + b1_ref[...], 0.0)
    t = jnp.dot(h1, w2_ref[...], preferred_element_type=_f32)
    dc = di[:, 0:1]
    hs2a_ref[...] = t[:, 0:16] * dc
    hs2b_ref[...] = t[:, 16:32] * dc


def _k4(hs1, s1a, s1b, dinvr, b1p, w2p):
    return pl.pallas_call(
        _k4_body,
        grid=(NP // _BLK,),
        in_specs=[
            pl.BlockSpec((_BLK, 16), lambda i: (i, 0)),
            pl.BlockSpec((_BLK, 16), lambda i: (i, 0)),
            pl.BlockSpec((_BLK, 16), lambda i: (i, 0)),
            pl.BlockSpec((_BLK, 16), lambda i: (i, 0)),
            pl.BlockSpec((1, 16), lambda i: (0, 0)),
            pl.BlockSpec((16, 32), lambda i: (0, 0)),
        ],
        out_specs=[
            pl.BlockSpec((_BLK, 16), lambda i: (i, 0)),
            pl.BlockSpec((_BLK, 16), lambda i: (i, 0)),
        ],
        out_shape=[
            jax.ShapeDtypeStruct((NP, 16), _f32),
            jax.ShapeDtypeStruct((NP, 16), _f32),
        ],
    )(hs1, s1a, s1b, dinvr, b1p, w2p)


def _k6_body(ga_ref, gb_ref, w0a_ref, w0b_ref, b0_ref, wf1_ref, bf1_ref,
             wf2_ref, bf2_ref, out_ref):
    t = (jnp.dot(ga_ref[...], w0a_ref[...], preferred_element_type=_f32)
         + jnp.dot(gb_ref[...], w0b_ref[...], preferred_element_type=_f32)
         + b0_ref[...])
    t = jnp.maximum(t, 0.0)
    t = jnp.maximum(
        jnp.dot(t, wf1_ref[...], preferred_element_type=_f32) + bf1_ref[...],
        0.0)
    out_ref[...] = (jnp.dot(t, wf2_ref[...], preferred_element_type=_f32)
                    + bf2_ref[...])


def _k6(ga, gb, w0a, w0b, b0p, wf1p, bf1p, wf2p, bf2r):
    return pl.pallas_call(
        _k6_body,
        out_shape=jax.ShapeDtypeStruct((G, 1), _f32),
    )(ga, gb, w0a, w0b, b0p, wf1p, bf1p, wf2p, bf2r)


def kernel(x, edge_index, batch, W1, b1, W2, b2, W0, b0, Wf1, bf1, Wf2, bf2):
    xp = jnp.pad(x, ((0, NP - N), (0, 5)))
    padidx = jnp.full((E2 - E,), NP - 1, _i32)
    src2 = jnp.concatenate([edge_index[0], padidx]).reshape(EROWS, 128)
    dst2 = jnp.concatenate([edge_index[1], padidx]).reshape(EROWS, 128)
    batch2 = jnp.pad(batch, (0, BROWS * 128 - N),
                     constant_values=G).reshape(BROWS, 128)
    zn = jnp.zeros((NP,), _f32)
    zn16 = jnp.zeros((NP, 16), _f32)
    w1p = jnp.pad(W1, ((0, 5), (0, 1)))
    b1p = jnp.pad(b1, (0, 1)).reshape(1, 16)
    w2p = jnp.pad(W2, ((0, 1), (0, 13)))
    b2ab = jnp.pad(b2, (0, 13))
    w0p = jnp.pad(W0, ((0, 13), (0, 6)))
    w0a, w0b = w0p[:16], w0p[16:]
    b0p = jnp.pad(b0, (0, 6)).reshape(1, 16)
    wf1p = jnp.pad(Wf1, ((0, 6), (0, 11)))
    bf1p = jnp.pad(bf1, (0, 11)).reshape(1, 16)
    wf2p = jnp.pad(Wf2, ((0, 11), (0, 0)))
    bf2r = bf2.reshape(1, 1)

    degpart, cnt = _k1(dst2, batch2, zn)
    pos = _kpos(cnt[:G].reshape(1, G)).reshape(G)
    degt = degpart.T
    hs1, dinvr = _k2(xp, degt, w1p)
    s1a, s1b = _k3(hs1, src2, dst2, zn16)
    hs2a, hs2b = _k4(hs1, s1a, s1b, dinvr, b1p, w2p)
    s2a0, s2a1 = _k3(hs2a, src2, dst2, zn16)
    s2b0, s2b1 = _k3(hs2b, src2, dst2, zn16)
    ga, gb = _k5(pos, dinvr, hs2a, hs2b, s2a0, s2a1, s2b0, s2b1, b2ab)
    return _k6(ga, gb, w0a, w0b, b0p, wf1p, bf1p, wf2p, bf2r)


# speedup vs baseline: 58.1116x; 1.7808x over previous
import functools

import jax
import jax.numpy as jnp
from jax import lax
from jax.experimental import pallas as pl
from jax.experimental.pallas import tpu as pltpu
from jax.experimental.pallas import tpu_sc as plsc

N = 100000
E = 3200000
G = 1024
GP = G + 128
NP = 102400
EROWS = 25280
E2 = EROWS * 128
BROWS = 896
NC = 2
NS = 16
NW = NC * NS
ROWS_PER_W = EROWS // NW
CHR = 5
NCHUNK = ROWS_PER_W // CHR
BR_PER_T = BROWS // NS

_mesh = plsc.VectorSubcoreMesh(core_axis_name="c", subcore_axis_name="s")
_f32 = jnp.float32
_i32 = jnp.int32


def _k1_body(dst_hbm, batch_hbm, zn_hbm, degout, cntout,
             idx_v, bidx_v, ones_v, sem, sh_deg, sh_cnt):
    c = lax.axis_index("c")
    s = lax.axis_index("s")
    wid = c * NS + s

    def _o(i, _):
        ones_v[pl.ds(i * 16, 16)] = jnp.ones((16,), _f32)
        return 0
    lax.fori_loop(0, 8, _o, 0)

    @pl.when(s == 0)
    def _():
        pltpu.sync_copy(zn_hbm, sh_deg)

    @pl.when(jnp.logical_and(c == 0, s == 1))
    def _():
        pltpu.sync_copy(zn_hbm.at[pl.ds(0, GP)], sh_cnt)

    plsc.subcore_barrier()

    def _chunk(ch, _):
        base = wid * ROWS_PER_W + ch * CHR
        pltpu.sync_copy(dst_hbm.at[pl.ds(base, CHR)], idx_v)
        ds = []
        for j in range(CHR):
            ds.append(pltpu.async_copy(
                ones_v, sh_deg.at[idx_v.at[j]], sem, add=True))
        for d in ds:
            d.wait()
        return 0
    lax.fori_loop(0, NCHUNK, _chunk, 0)

    @pl.when(c == 0)
    def _():
        pltpu.sync_copy(batch_hbm.at[pl.ds(s * BR_PER_T, BR_PER_T)], bidx_v)
        for t in range(BR_PER_T // 4):
            ds = []
            for j in range(4):
                ds.append(pltpu.async_copy(
                    ones_v, sh_cnt.at[bidx_v.at[t * 4 + j]], sem, add=True))
            for d in ds:
                d.wait()

    plsc.subcore_barrier()

    @pl.when(s == 0)
    def _():
        pltpu.sync_copy(sh_deg, degout.at[c])

    @pl.when(jnp.logical_and(c == 0, s == 1))
    def _():
        pltpu.sync_copy(sh_cnt, cntout)


_k1 = functools.partial(
    pl.kernel,
    out_type=(jax.ShapeDtypeStruct((NC, NP), _f32),
              jax.ShapeDtypeStruct((GP,), _f32)),
    mesh=_mesh,
    compiler_params=pltpu.CompilerParams(use_tc_tiling_on_sc=False),
    scratch_types=[
        pltpu.VMEM((CHR, 128), _i32),
        pltpu.VMEM((BR_PER_T, 128), _i32),
        pltpu.VMEM((128,), _f32),
        pltpu.SemaphoreType.DMA,
        pltpu.VMEM_SHARED((NP,), _f32),
        pltpu.VMEM_SHARED((GP,), _f32),
    ],
)(_k1_body)


def _k3_body(hs_hbm, src_hbm, dst_hbm, zn16_hbm, outa, outb,
             idx_sa, idx_da, idx_sb, idx_db, rows_a, rows_b,
             gsa, gsb, ssa, ssb, s_sh):
    c = lax.axis_index("c")
    s = lax.axis_index("s")
    wid = c * NS + s

    @pl.when(s == 0)
    def _():
        pltpu.sync_copy(zn16_hbm, s_sh)

    plsc.subcore_barrier()
    w0 = wid * ROWS_PER_W

    def _stage(ch, isx, idx):
        pltpu.sync_copy(src_hbm.at[pl.ds(w0 + ch * CHR, CHR)], isx)
        pltpu.sync_copy(dst_hbm.at[pl.ds(w0 + ch * CHR, CHR)], idx)

    def _gfire(isx, rows, sem):
        for j in range(CHR):
            pltpu.async_copy(hs_hbm.at[isx.at[j]],
                             rows.at[pl.ds(j * 128, 128)], sem)

    def _gwait(isx, rows, sem):
        for j in range(CHR):
            pltpu.make_async_copy(hs_hbm.at[isx.at[j]],
                                  rows.at[pl.ds(j * 128, 128)], sem).wait()

    def _sfire(idx, rows, sem):
        for j in range(CHR):
            pltpu.async_copy(rows.at[pl.ds(j * 128, 128)],
                             s_sh.at[idx.at[j]], sem, add=True)

    def _swait(idx, rows, sem):
        for j in range(CHR):
            pltpu.make_async_copy(rows.at[pl.ds(j * 128, 128)],
                                  s_sh.at[idx.at[j]], sem).wait()

    _stage(0, idx_sa, idx_da)
    _gfire(idx_sa, rows_a, gsa)
    _stage(1, idx_sb, idx_db)
    _gwait(idx_sa, rows_a, gsa)
    _sfire(idx_da, rows_a, ssa)
    _gfire(idx_sb, rows_b, gsb)

    def _steady(ch, _):
        @pl.when(ch % 2 == 0)
        def _():
            _swait(idx_db, rows_b, ssb)
            _stage(ch + 1, idx_sb, idx_db)
            _gwait(idx_sa, rows_a, gsa)
            _gfire(idx_sb, rows_b, gsb)
            _sfire(idx_da, rows_a, ssa)

        @pl.when(ch % 2 == 1)
        def _():
            _swait(idx_da, rows_a, ssa)
            _stage(ch + 1, idx_sa, idx_da)
            _gwait(idx_sb, rows_b, gsb)
            _gfire(idx_sa, rows_a, gsa)
            _sfire(idx_db, rows_b, ssb)
        return 0
    lax.fori_loop(1, NCHUNK - 1, _steady, 0)

    _swait(idx_da, rows_a, ssa)
    _gwait(idx_sb, rows_b, gsb)
    _sfire(idx_db, rows_b, ssb)
    _swait(idx_db, rows_b, ssb)
    plsc.subcore_barrier()

    @pl.when(jnp.logical_and(c == 0, s == 0))
    def _():
        pltpu.sync_copy(s_sh, outa)

    @pl.when(jnp.logical_and(c == 1, s == 0))
    def _():
        pltpu.sync_copy(s_sh, outb)


_k3 = functools.partial(
    pl.kernel,
    out_type=(jax.ShapeDtypeStruct((NP, 16), _f32),
              jax.ShapeDtypeStruct((NP, 16), _f32)),
    mesh=_mesh,
    compiler_params=pltpu.CompilerParams(use_tc_tiling_on_sc=False),
    scratch_types=[
        pltpu.VMEM((CHR, 128), _i32),
        pltpu.VMEM((CHR, 128), _i32),
        pltpu.VMEM((CHR, 128), _i32),
        pltpu.VMEM((CHR, 128), _i32),
        pltpu.VMEM((CHR * 128, 16), _f32),
        pltpu.VMEM((CHR * 128, 16), _f32),
        pltpu.SemaphoreType.DMA,
        pltpu.SemaphoreType.DMA,
        pltpu.SemaphoreType.DMA,
        pltpu.SemaphoreType.DMA,
        pltpu.VMEM_SHARED((NP, 16), _f32),
    ],
)(_k3_body)


def _k5_body(pos_hbm, dinvr_hbm, hs2a_hbm, hs2b_hbm, sa0_h, sa1_h, sb0_h, sb1_h,
             b2_hbm, ga_out, gb_out,
             idxv, dbuf, ba, bb, a0, a1, b0_, b1_, biasv, gav, gbv, sem):
    c = lax.axis_index("c")
    s = lax.axis_index("s")
    wid = c * NS + s
    base = wid * 32
    pltpu.sync_copy(pos_hbm.at[pl.ds(base, 32)], idxv)
    pltpu.sync_copy(b2_hbm, biasv)
    gd = []
    for tbl, buf in ((dinvr_hbm, dbuf), (hs2a_hbm, ba), (hs2b_hbm, bb),
                     (sa0_h, a0), (sa1_h, a1), (sb0_h, b0_), (sb1_h, b1_)):
        gd.append(pltpu.async_copy(tbl.at[idxv], buf, sem))
    for d in gd:
        d.wait()
    b2a = biasv[pl.ds(0, 16)]
    b2b = biasv[pl.ds(16, 16)]
    for r in range(32):
        di = dbuf[r]
        ha = jnp.maximum(di * (ba[r] + a0[r] + a1[r]) + b2a, 0.0)
        hb = jnp.maximum(di * (bb[r] + b0_[r] + b1_[r]) + b2b, 0.0)
        gav[r] = ha
        gbv[r] = hb
    pltpu.sync_copy(gav, ga_out.at[pl.ds(base, 32)])
    pltpu.sync_copy(gbv, gb_out.at[pl.ds(base, 32)])


_k5 = functools.partial(
    pl.kernel,
    out_type=(jax.ShapeDtypeStruct((G, 16), _f32),
              jax.ShapeDtypeStruct((G, 16), _f32)),
    mesh=_mesh,
    compiler_params=pltpu.CompilerParams(use_tc_tiling_on_sc=False),
    scratch_types=[
        pltpu.VMEM((32,), _i32),
        pltpu.VMEM((32, 16), _f32),
        pltpu.VMEM((32, 16), _f32),
        pltpu.VMEM((32, 16), _f32),
        pltpu.VMEM((32, 16), _f32),
        pltpu.VMEM((32, 16), _f32),
        pltpu.VMEM((32, 16), _f32),
        pltpu.VMEM((32, 16), _f32),
        pltpu.VMEM((32,), _f32),
        pltpu.VMEM((32, 16), _f32),
        pltpu.VMEM((32, 16), _f32),
        pltpu.SemaphoreType.DMA,
    ],
)(_k5_body)


_BLK = 1024


def _kpos_body(cnt_ref, pos_ref):
    r = lax.broadcasted_iota(_i32, (G, G), 0)
    col = lax.broadcasted_iota(_i32, (G, G), 1)
    tri = (r <= col).astype(_f32)
    cum = jnp.dot(cnt_ref[...], tri, preferred_element_type=_f32)
    p = cum.astype(_i32) - 1
    pos_ref[...] = jnp.where(p < 0, N - 1, p)


def _kpos(cnt):
    return pl.pallas_call(
        _kpos_body,
        out_shape=jax.ShapeDtypeStruct((1, G), _i32),
    )(cnt)


def _k2_body(x_ref, degt_ref, w1_ref, hs1_ref, dinvr_ref):
    deg = degt_ref[:, 0:1] + degt_ref[:, 1:2] + 1.0
    dinv = lax.rsqrt(deg)
    h = jnp.dot(x_ref[...], w1_ref[...], preferred_element_type=_f32)
    hs1_ref[...] = h * dinv
    dinvr_ref[...] = jnp.broadcast_to(dinv, (_BLK, 16))


def _k2(xp, degt, w1p):
    return pl.pallas_call(
        _k2_body,
        grid=(NP // _BLK,),
        in_specs=[
            pl.BlockSpec((_BLK, 16), lambda i: (i, 0)),
            pl.BlockSpec((_BLK, 2), lambda i: (i, 0)),
            pl.BlockSpec((16, 16), lambda i: (0, 0)),
        ],
        out_specs=[
            pl.BlockSpec((_BLK, 16), lambda i: (i, 0)),
            pl.BlockSpec((_BLK, 16), lambda i: (i, 0)),
        ],
        out_shape=[
            jax.ShapeDtypeStruct((NP, 16), _f32),
            jax.ShapeDtypeStruct((NP, 16), _f32),
        ],
    )(xp, degt, w1p)


def _k4_body(hs1_ref, s1a_ref, s1b_ref, dinvr_ref, b1_ref, w2_ref,
             hs2a_ref, hs2b_ref):
    di = dinvr_ref[...]
    h1 = jnp.maximum(
        di * (hs1_ref[...] + s1a_ref[...] + s1b_ref[...]) + b1_ref[...], 0.0)
    t = jnp.dot(h1, w2_ref[...], preferred_element_type=_f32)
    dc = di[:, 0:1]
    hs2a_ref[...] = t[:, 0:16] * dc
    hs2b_ref[...] = t[:, 16:32] * dc


def _k4(hs1, s1a, s1b, dinvr, b1p, w2p):
    return pl.pallas_call(
        _k4_body,
        grid=(NP // _BLK,),
        in_specs=[
            pl.BlockSpec((_BLK, 16), lambda i: (i, 0)),
            pl.BlockSpec((_BLK, 16), lambda i: (i, 0)),
            pl.BlockSpec((_BLK, 16), lambda i: (i, 0)),
            pl.BlockSpec((_BLK, 16), lambda i: (i, 0)),
            pl.BlockSpec((1, 16), lambda i: (0, 0)),
            pl.BlockSpec((16, 32), lambda i: (0, 0)),
        ],
        out_specs=[
            pl.BlockSpec((_BLK, 16), lambda i: (i, 0)),
            pl.BlockSpec((_BLK, 16), lambda i: (i, 0)),
        ],
        out_shape=[
            jax.ShapeDtypeStruct((NP, 16), _f32),
            jax.ShapeDtypeStruct((NP, 16), _f32),
        ],
    )(hs1, s1a, s1b, dinvr, b1p, w2p)


def _k6_body(ga_ref, gb_ref, w0a_ref, w0b_ref, b0_ref, wf1_ref, bf1_ref,
             wf2_ref, bf2_ref, out_ref):
    t = (jnp.dot(ga_ref[...], w0a_ref[...], preferred_element_type=_f32)
         + jnp.dot(gb_ref[...], w0b_ref[...], preferred_element_type=_f32)
         + b0_ref[...])
    t = jnp.maximum(t, 0.0)
    t = jnp.maximum(
        jnp.dot(t, wf1_ref[...], preferred_element_type=_f32) + bf1_ref[...],
        0.0)
    out_ref[...] = (jnp.dot(t, wf2_ref[...], preferred_element_type=_f32)
                    + bf2_ref[...])


def _k6(ga, gb, w0a, w0b, b0p, wf1p, bf1p, wf2p, bf2r):
    return pl.pallas_call(
        _k6_body,
        out_shape=jax.ShapeDtypeStruct((G, 1), _f32),
    )(ga, gb, w0a, w0b, b0p, wf1p, bf1p, wf2p, bf2r)


def kernel(x, edge_index, batch, W1, b1, W2, b2, W0, b0, Wf1, bf1, Wf2, bf2):
    xp = jnp.pad(x, ((0, NP - N), (0, 5)))
    padidx = N + (jnp.arange(E2 - E, dtype=_i32) % (NP - N))
    src2 = jnp.concatenate([edge_index[0], padidx]).reshape(EROWS, 128)
    dst2 = jnp.concatenate([edge_index[1], padidx]).reshape(EROWS, 128)
    batch2 = jnp.pad(batch, (0, BROWS * 128 - N),
                     constant_values=G).reshape(BROWS, 128)
    zn = jnp.zeros((NP,), _f32)
    zn16 = jnp.zeros((NP, 16), _f32)
    w1p = jnp.pad(W1, ((0, 5), (0, 1)))
    b1p = jnp.pad(b1, (0, 1)).reshape(1, 16)
    w2p = jnp.pad(W2, ((0, 1), (0, 13)))
    b2ab = jnp.pad(b2, (0, 13))
    w0p = jnp.pad(W0, ((0, 13), (0, 6)))
    w0a, w0b = w0p[:16], w0p[16:]
    b0p = jnp.pad(b0, (0, 6)).reshape(1, 16)
    wf1p = jnp.pad(Wf1, ((0, 6), (0, 11)))
    bf1p = jnp.pad(bf1, (0, 11)).reshape(1, 16)
    wf2p = jnp.pad(Wf2, ((0, 11), (0, 0)))
    bf2r = bf2.reshape(1, 1)

    degpart, cnt = _k1(dst2, batch2, zn)
    pos = _kpos(cnt[:G].reshape(1, G)).reshape(G)
    degt = degpart.T
    hs1, dinvr = _k2(xp, degt, w1p)
    s1a, s1b = _k3(hs1, src2, dst2, zn16)
    hs2a, hs2b = _k4(hs1, s1a, s1b, dinvr, b1p, w2p)
    s2a0, s2a1 = _k3(hs2a, src2, dst2, zn16)
    s2b0, s2b1 = _k3(hs2b, src2, dst2, zn16)
    ga, gb = _k5(pos, dinvr, hs2a, hs2b, s2a0, s2a1, s2b0, s2b1, b2ab)
    return _k6(ga, gb, w0a, w0b, b0p, wf1p, bf1p, wf2p, bf2r)


# speedup vs baseline: 72.5591x; 1.2486x over previous
import functools

import jax
import jax.numpy as jnp
from jax import lax
from jax.experimental import pallas as pl
from jax.experimental.pallas import tpu as pltpu
from jax.experimental.pallas import tpu_sc as plsc

N = 100000
E = 3200000
G = 1024
GP = G + 128
NP = 102400
EROWS = 25280
E2 = EROWS * 128
BROWS = 896
NC = 2
NS = 16
NW = NC * NS
ROWS_PER_W = EROWS // NW
CHR = 5
NCHUNK = ROWS_PER_W // CHR
BR_PER_T = BROWS // NS

_mesh = plsc.VectorSubcoreMesh(core_axis_name="c", subcore_axis_name="s")
_f32 = jnp.float32
_i32 = jnp.int32


def _k1_body(dst_hbm, batch_hbm, zn_hbm, degout, cntout,
             idx_v, bidx_v, ones_v, sem, sh_deg, sh_cnt):
    c = lax.axis_index("c")
    s = lax.axis_index("s")
    wid = c * NS + s

    def _o(i, _):
        ones_v[pl.ds(i * 16, 16)] = jnp.ones((16,), _f32)
        return 0
    lax.fori_loop(0, 8, _o, 0)

    @pl.when(s == 0)
    def _():
        pltpu.sync_copy(zn_hbm, sh_deg)

    @pl.when(jnp.logical_and(c == 0, s == 1))
    def _():
        pltpu.sync_copy(zn_hbm.at[pl.ds(0, GP)], sh_cnt)

    plsc.subcore_barrier()

    def _chunk(ch, _):
        base = wid * ROWS_PER_W + ch * CHR
        pltpu.sync_copy(dst_hbm.at[pl.ds(base, CHR)], idx_v)
        ds = []
        for j in range(CHR):
            ds.append(pltpu.async_copy(
                ones_v, sh_deg.at[idx_v.at[j]], sem, add=True))
        for d in ds:
            d.wait()
        return 0
    lax.fori_loop(0, NCHUNK, _chunk, 0)

    @pl.when(c == 0)
    def _():
        pltpu.sync_copy(batch_hbm.at[pl.ds(s * BR_PER_T, BR_PER_T)], bidx_v)
        for t in range(BR_PER_T // 4):
            ds = []
            for j in range(4):
                ds.append(pltpu.async_copy(
                    ones_v, sh_cnt.at[bidx_v.at[t * 4 + j]], sem, add=True))
            for d in ds:
                d.wait()

    plsc.subcore_barrier()

    @pl.when(s == 0)
    def _():
        pltpu.sync_copy(sh_deg, degout.at[c])

    @pl.when(jnp.logical_and(c == 0, s == 1))
    def _():
        pltpu.sync_copy(sh_cnt, cntout)


_k1 = functools.partial(
    pl.kernel,
    out_type=(jax.ShapeDtypeStruct((NC, NP), _f32),
              jax.ShapeDtypeStruct((GP,), _f32)),
    mesh=_mesh,
    compiler_params=pltpu.CompilerParams(use_tc_tiling_on_sc=False),
    scratch_types=[
        pltpu.VMEM((CHR, 128), _i32),
        pltpu.VMEM((BR_PER_T, 128), _i32),
        pltpu.VMEM((128,), _f32),
        pltpu.SemaphoreType.DMA,
        pltpu.VMEM_SHARED((NP,), _f32),
        pltpu.VMEM_SHARED((GP,), _f32),
    ],
)(_k1_body)


def _k3_body(hs_hbm, src_hbm, dst_hbm, zn16_hbm, outa, outb,
             idx_sa, idx_da, idx_sb, idx_db, rows_a, rows_b,
             gsa, gsb, ssa, ssb, s_sh):
    c = lax.axis_index("c")
    s = lax.axis_index("s")
    wid = c * NS + s

    @pl.when(s == 0)
    def _():
        pltpu.sync_copy(zn16_hbm, s_sh)

    plsc.subcore_barrier()
    w0 = wid * ROWS_PER_W

    def _stage(ch, isx, idx):
        pltpu.sync_copy(src_hbm.at[pl.ds(w0 + ch * CHR, CHR)], isx)
        pltpu.sync_copy(dst_hbm.at[pl.ds(w0 + ch * CHR, CHR)], idx)

    def _gfire(isx, rows, sem):
        for j in range(CHR):
            pltpu.async_copy(hs_hbm.at[isx.at[j]],
                             rows.at[pl.ds(j * 128, 128)], sem)

    def _gwait(isx, rows, sem):
        for j in range(CHR):
            pltpu.make_async_copy(hs_hbm.at[isx.at[j]],
                                  rows.at[pl.ds(j * 128, 128)], sem).wait()

    def _sfire(idx, rows, sem):
        for j in range(CHR):
            pltpu.async_copy(rows.at[pl.ds(j * 128, 128)],
                             s_sh.at[idx.at[j]], sem, add=True)

    def _swait(idx, rows, sem):
        for j in range(CHR):
            pltpu.make_async_copy(rows.at[pl.ds(j * 128, 128)],
                                  s_sh.at[idx.at[j]], sem).wait()

    _stage(0, idx_sa, idx_da)
    _gfire(idx_sa, rows_a, gsa)
    _stage(1, idx_sb, idx_db)
    _gwait(idx_sa, rows_a, gsa)
    _sfire(idx_da, rows_a, ssa)
    _gfire(idx_sb, rows_b, gsb)

    def _steady(ch, _):
        @pl.when(ch % 2 == 0)
        def _():
            _swait(idx_db, rows_b, ssb)
            _stage(ch + 1, idx_sb, idx_db)
            _gwait(idx_sa, rows_a, gsa)
            _gfire(idx_sb, rows_b, gsb)
            _sfire(idx_da, rows_a, ssa)

        @pl.when(ch % 2 == 1)
        def _():
            _swait(idx_da, rows_a, ssa)
            _stage(ch + 1, idx_sa, idx_da)
            _gwait(idx_sb, rows_b, gsb)
            _gfire(idx_sa, rows_a, gsa)
            _sfire(idx_db, rows_b, ssb)
        return 0
    lax.fori_loop(1, NCHUNK - 1, _steady, 0)

    _swait(idx_da, rows_a, ssa)
    _gwait(idx_sb, rows_b, gsb)
    _sfire(idx_db, rows_b, ssb)
    _swait(idx_db, rows_b, ssb)
    plsc.subcore_barrier()

    @pl.when(jnp.logical_and(c == 0, s == 0))
    def _():
        pltpu.sync_copy(s_sh, outa)

    @pl.when(jnp.logical_and(c == 1, s == 0))
    def _():
        pltpu.sync_copy(s_sh, outb)


_k3 = functools.partial(
    pl.kernel,
    out_type=(jax.ShapeDtypeStruct((NP, 16), _f32),
              jax.ShapeDtypeStruct((NP, 16), _f32)),
    mesh=_mesh,
    compiler_params=pltpu.CompilerParams(use_tc_tiling_on_sc=False),
    scratch_types=[
        pltpu.VMEM((CHR, 128), _i32),
        pltpu.VMEM((CHR, 128), _i32),
        pltpu.VMEM((CHR, 128), _i32),
        pltpu.VMEM((CHR, 128), _i32),
        pltpu.VMEM((CHR * 128, 16), _f32),
        pltpu.VMEM((CHR * 128, 16), _f32),
        pltpu.SemaphoreType.DMA,
        pltpu.SemaphoreType.DMA,
        pltpu.SemaphoreType.DMA,
        pltpu.SemaphoreType.DMA,
        pltpu.VMEM_SHARED((NP, 16), _f32),
    ],
)(_k3_body)


def _k5_body(pos_hbm, dinvr_hbm, u_hbm, s0_h, s1_h, g_out,
             idxv, dbuf, ub, s0b, s1b, gv, sem):
    c = lax.axis_index("c")
    s = lax.axis_index("s")
    wid = c * NS + s
    base = wid * 32
    pltpu.sync_copy(pos_hbm.at[pl.ds(base, 32)], idxv)
    gd = []
    for tbl, buf in ((dinvr_hbm, dbuf), (u_hbm, ub), (s0_h, s0b), (s1_h, s1b)):
        gd.append(pltpu.async_copy(tbl.at[idxv], buf, sem))
    for d in gd:
        d.wait()
    for r in range(32):
        gv[r] = dbuf[r] * (ub[r] + s0b[r] + s1b[r])
    pltpu.sync_copy(gv, g_out.at[pl.ds(base, 32)])


_k5 = functools.partial(
    pl.kernel,
    out_type=jax.ShapeDtypeStruct((G, 16), _f32),
    mesh=_mesh,
    compiler_params=pltpu.CompilerParams(use_tc_tiling_on_sc=False),
    scratch_types=[
        pltpu.VMEM((32,), _i32),
        pltpu.VMEM((32, 16), _f32),
        pltpu.VMEM((32, 16), _f32),
        pltpu.VMEM((32, 16), _f32),
        pltpu.VMEM((32, 16), _f32),
        pltpu.VMEM((32, 16), _f32),
        pltpu.SemaphoreType.DMA,
    ],
)(_k5_body)


_BLK = 1024


def _kpos_body(cnt_ref, pos_ref):
    r = lax.broadcasted_iota(_i32, (G, G), 0)
    col = lax.broadcasted_iota(_i32, (G, G), 1)
    tri = (r <= col).astype(_f32)
    cum = jnp.dot(cnt_ref[...], tri, preferred_element_type=_f32,
            precision=lax.Precision.HIGHEST)
    p = cum.astype(_i32) - 1
    pos_ref[...] = jnp.where(p < 0, N - 1, p)


def _kpos(cnt):
    return pl.pallas_call(
        _kpos_body,
        out_shape=jax.ShapeDtypeStruct((1, G), _i32),
    )(cnt)


def _k2_body(x_ref, degt_ref, w1_ref, hs1_ref, dinvr_ref):
    deg = degt_ref[:, 0:1] + degt_ref[:, 1:2] + 1.0
    dinv = lax.rsqrt(deg)
    h = jnp.dot(x_ref[...], w1_ref[...], preferred_element_type=_f32,
            precision=lax.Precision.HIGHEST)
    hs1_ref[...] = h * dinv
    dinvr_ref[...] = jnp.broadcast_to(dinv, (_BLK, 16))


def _k2(xp, degt, w1p):
    return pl.pallas_call(
        _k2_body,
        grid=(NP // _BLK,),
        in_specs=[
            pl.BlockSpec((_BLK, 16), lambda i: (i, 0)),
            pl.BlockSpec((_BLK, 2), lambda i: (i, 0)),
            pl.BlockSpec((16, 16), lambda i: (0, 0)),
        ],
        out_specs=[
            pl.BlockSpec((_BLK, 16), lambda i: (i, 0)),
            pl.BlockSpec((_BLK, 16), lambda i: (i, 0)),
        ],
        out_shape=[
            jax.ShapeDtypeStruct((NP, 16), _f32),
            jax.ShapeDtypeStruct((NP, 16), _f32),
        ],
    )(xp, degt, w1p)


def _k4_body(hs1_ref, s1a_ref, s1b_ref, dinvr_ref, b1_ref, u_ref):
    di = dinvr_ref[...]
    h1 = jnp.maximum(
        di * (hs1_ref[...] + s1a_ref[...] + s1b_ref[...]) + b1_ref[...], 0.0)
    u_ref[...] = h1 * di


def _k4(hs1, s1a, s1b, dinvr, b1p):
    return pl.pallas_call(
        _k4_body,
        grid=(NP // _BLK,),
        in_specs=[
            pl.BlockSpec((_BLK, 16), lambda i: (i, 0)),
            pl.BlockSpec((_BLK, 16), lambda i: (i, 0)),
            pl.BlockSpec((_BLK, 16), lambda i: (i, 0)),
            pl.BlockSpec((_BLK, 16), lambda i: (i, 0)),
            pl.BlockSpec((1, 16), lambda i: (0, 0)),
        ],
        out_specs=pl.BlockSpec((_BLK, 16), lambda i: (i, 0)),
        out_shape=jax.ShapeDtypeStruct((NP, 16), _f32),
    )(hs1, s1a, s1b, dinvr, b1p)


def _k6_body(v_ref, w2_ref, b2_ref, w0_ref, b0_ref, wf1_ref, bf1_ref,
             wf2_ref, bf2_ref, out_ref):
    h2 = jnp.maximum(
        jnp.dot(v_ref[...], w2_ref[...], preferred_element_type=_f32,
            precision=lax.Precision.HIGHEST)
        + b2_ref[...], 0.0)
    t = jnp.maximum(
        jnp.dot(h2, w0_ref[...], preferred_element_type=_f32,
            precision=lax.Precision.HIGHEST) + b0_ref[...],
        0.0)
    t = jnp.maximum(
        jnp.dot(t, wf1_ref[...], preferred_element_type=_f32,
            precision=lax.Precision.HIGHEST) + bf1_ref[...],
        0.0)
    out_ref[...] = (jnp.dot(t, wf2_ref[...], preferred_element_type=_f32,
            precision=lax.Precision.HIGHEST)
                    + bf2_ref[...])


def _k6(v, w2p, b2p, w0p, b0p, wf1p, bf1p, wf2p, bf2r):
    return pl.pallas_call(
        _k6_body,
        out_shape=jax.ShapeDtypeStruct((G, 1), _f32),
    )(v, w2p, b2p, w0p, b0p, wf1p, bf1p, wf2p, bf2r)


def kernel(x, edge_index, batch, W1, b1, W2, b2, W0, b0, Wf1, bf1, Wf2, bf2):
    xp = jnp.pad(x, ((0, NP - N), (0, 5)))
    padidx = N + (jnp.arange(E2 - E, dtype=_i32) % (NP - N))
    src2 = jnp.concatenate([edge_index[0], padidx]).reshape(EROWS, 128)
    dst2 = jnp.concatenate([edge_index[1], padidx]).reshape(EROWS, 128)
    batch2 = jnp.pad(batch, (0, BROWS * 128 - N),
                     constant_values=G).reshape(BROWS, 128)
    zn = jnp.zeros((NP,), _f32)
    zn16 = jnp.zeros((NP, 16), _f32)
    w1p = jnp.pad(W1, ((0, 5), (0, 1)))
    b1p = jnp.pad(b1, (0, 1)).reshape(1, 16)
    w2p = jnp.pad(W2, ((0, 1), (0, 13)))
    b2p = jnp.pad(b2, (0, 13)).reshape(1, 32)
    w0p = jnp.pad(W0, ((0, 13), (0, 6)))
    b0p = jnp.pad(b0, (0, 6)).reshape(1, 16)
    wf1p = jnp.pad(Wf1, ((0, 6), (0, 11)))
    bf1p = jnp.pad(bf1, (0, 11)).reshape(1, 16)
    wf2p = jnp.pad(Wf2, ((0, 11), (0, 0)))
    bf2r = bf2.reshape(1, 1)

    degpart, cnt = _k1(dst2, batch2, zn)
    pos = _kpos(cnt[:G].reshape(1, G)).reshape(G)
    degt = degpart.T
    hs1, dinvr = _k2(xp, degt, w1p)
    s1a, s1b = _k3(hs1, src2, dst2, zn16)
    u = _k4(hs1, s1a, s1b, dinvr, b1p)
    s2a, s2b = _k3(u, src2, dst2, zn16)
    v = _k5(pos, dinvr, u, s2a, s2b)
    return _k6(v, w2p, b2p, w0p, b0p, wf1p, bf1p, wf2p, bf2r)
